# Initial kernel scaffold; baseline (speedup 1.0000x reference)
import jax
import jax.numpy as jnp
from jax.experimental import pallas as pl


def kernel(x, edge_index, W1, b1, W2, b2):
    raise NotImplementedError("write your pallas kernel here")


# speedup vs baseline: 18.6133x; 18.6133x over previous
import functools

import jax
import jax.numpy as jnp
from jax import lax
from jax.experimental import pallas as pl
from jax.experimental.pallas import tpu as pltpu
from jax.experimental.pallas import tpu_sc as plsc

N = 10000
E = 320000
D = 128
NC = 2
NS = 16
NW = NC * NS
EPW = E // NW
CH = 80
NCHUNK = EPW // CH
N2 = 10240
RPT = N2 // NS
ZR = 40
DEGW = 16
BLK = 400
GRID = N // BLK

_mesh = plsc.VectorSubcoreMesh(
    core_axis_name="c", subcore_axis_name="s", num_cores=NC, num_subcores=NS
)


@functools.partial(
    pl.kernel,
    out_type=jax.ShapeDtypeStruct((NC, N2, DEGW), jnp.float32),
    mesh=_mesh,
    scratch_types=[
        pltpu.VMEM((NCHUNK, 1, CH), jnp.int32),
        pltpu.VMEM((CH, DEGW), jnp.float32),
        pltpu.VMEM((ZR, DEGW), jnp.float32),
        pltpu.VMEM_SHARED((N2, DEGW), jnp.float32),
    ],
)
def _sc_deg(dst4_hbm, out_hbm, idx_v, ones_v, zer_v, acc):
    c = lax.axis_index("c")
    s = lax.axis_index("s")
    w = c * NS + s

    def fill_ones(i, carry):
        ones_v[i, :] = jnp.ones((16,), jnp.float32)
        return carry

    lax.fori_loop(0, CH, fill_ones, 0)

    def fill_zero(i, carry):
        zer_v[i, :] = jnp.zeros((16,), jnp.float32)
        return carry

    lax.fori_loop(0, ZR, fill_zero, 0)

    for k in range(RPT // ZR):
        pltpu.sync_copy(zer_v, acc.at[pl.ds(s * RPT + k * ZR, ZR)])
    plsc.subcore_barrier()

    pltpu.sync_copy(dst4_hbm.at[w], idx_v)

    def body(j, carry):
        pltpu.sync_copy(ones_v, acc.at[idx_v.at[j, 0]], add=True)
        return carry

    lax.fori_loop(0, NCHUNK, body, 0)
    plsc.subcore_barrier()
    pltpu.sync_copy(acc.at[pl.ds(s * RPT, RPT)], out_hbm.at[c, pl.ds(s * RPT, RPT)])


@functools.partial(
    pl.kernel,
    out_type=jax.ShapeDtypeStruct((NC, N2, D), jnp.float32),
    mesh=_mesh,
    scratch_types=[
        pltpu.VMEM((NCHUNK, 1, CH), jnp.int32),
        pltpu.VMEM((NCHUNK, 1, CH), jnp.int32),
        pltpu.VMEM((CH, D), jnp.float32),
        pltpu.VMEM((ZR, D), jnp.float32),
        pltpu.VMEM_SHARED((N2, D), jnp.float32),
        pltpu.SemaphoreType.DMA,
    ],
)
def _sc_agg(g_hbm, src4_hbm, dst4_hbm, out_hbm, idxs_v, idxd_v, rows_v, zer_v, acc, sem):
    c = lax.axis_index("c")
    s = lax.axis_index("s")
    w = c * NS + s

    def fill_zero(i, carry):
        r = i // (D // 16)
        k = i - r * (D // 16)
        zer_v[r, pl.ds(k * 16, 16)] = jnp.zeros((16,), jnp.float32)
        return carry

    lax.fori_loop(0, ZR * (D // 16), fill_zero, 0)

    for k in range(RPT // ZR):
        pltpu.sync_copy(zer_v, acc.at[pl.ds(s * RPT + k * ZR, ZR)])
    plsc.subcore_barrier()

    pltpu.sync_copy(src4_hbm.at[w], idxs_v)
    pltpu.sync_copy(dst4_hbm.at[w], idxd_v)

    def body(j, carry):
        pltpu.async_copy(g_hbm.at[idxs_v.at[j, 0]], rows_v, sem).wait()
        pltpu.sync_copy(rows_v, acc.at[idxd_v.at[j, 0]], add=True)
        return carry

    lax.fori_loop(0, NCHUNK, body, 0)
    plsc.subcore_barrier()
    pltpu.sync_copy(acc.at[pl.ds(s * RPT, RPT)], out_hbm.at[c, pl.ds(s * RPT, RPT)])


def _dis(deg_ref):
    return lax.rsqrt(1.0 + deg_ref[0, :, 0:1] + deg_ref[1, :, 0:1])


def _tc_g1_body(x_ref, w_ref, deg_ref, o_ref):
    h = jnp.dot(x_ref[...], w_ref[...], preferred_element_type=jnp.float32)
    o_ref[...] = h * _dis(deg_ref)


def _tc_g2_body(p_ref, g_ref, deg_ref, b_ref, w_ref, o_ref):
    dis = _dis(deg_ref)
    ssum = p_ref[0] + p_ref[1] + g_ref[...]
    h = jnp.maximum(dis * ssum + b_ref[...], 0.0)
    o_ref[...] = jnp.dot(h, w_ref[...], preferred_element_type=jnp.float32) * dis


def _tc_out_body(p_ref, g_ref, deg_ref, b_ref, o_ref):
    dis = _dis(deg_ref)
    o_ref[...] = dis * (p_ref[0] + p_ref[1] + g_ref[...]) + b_ref[...]


_row_spec = pl.BlockSpec((BLK, D), lambda i: (i, 0))
_w_spec = pl.BlockSpec((D, D), lambda i: (0, 0))
_deg_spec = pl.BlockSpec((NC, BLK, DEGW), lambda i: (0, i, 0))
_p_spec = pl.BlockSpec((NC, BLK, D), lambda i: (0, i, 0))
_b_spec = pl.BlockSpec((1, D), lambda i: (0, 0))

_g1_call = pl.pallas_call(
    _tc_g1_body,
    grid=(GRID,),
    in_specs=[_row_spec, _w_spec, _deg_spec],
    out_specs=_row_spec,
    out_shape=jax.ShapeDtypeStruct((N, D), jnp.float32),
)

_g2_call = pl.pallas_call(
    _tc_g2_body,
    grid=(GRID,),
    in_specs=[_p_spec, _row_spec, _deg_spec, _b_spec, _w_spec],
    out_specs=_row_spec,
    out_shape=jax.ShapeDtypeStruct((N, D), jnp.float32),
)

_out_call = pl.pallas_call(
    _tc_out_body,
    grid=(GRID,),
    in_specs=[_p_spec, _row_spec, _deg_spec, _b_spec],
    out_specs=_row_spec,
    out_shape=jax.ShapeDtypeStruct((N, D), jnp.float32),
)


def kernel(x, edge_index, W1, b1, W2, b2):
    src4 = edge_index[0].reshape(NW, NCHUNK, 1, CH)
    dst4 = edge_index[1].reshape(NW, NCHUNK, 1, CH)
    b1r = b1.reshape(1, D)
    b2r = b2.reshape(1, D)

    deg16 = _sc_deg(dst4)
    g1 = _g1_call(x, W1, deg16)
    p1 = _sc_agg(g1, src4, dst4)
    g2 = _g2_call(p1, g1, deg16, b1r, W2)
    p2 = _sc_agg(g2, src4, dst4)
    out = _out_call(p2, g2, deg16, b2r)
    return out
